# baseline (speedup 1.0000x reference)
import functools

import jax
import jax.numpy as jnp
from jax import lax
from jax.experimental import pallas as pl
from jax.experimental.pallas import tpu as pltpu
from jax.experimental.pallas import tpu_sc as plsc

_B = 16384
_D = 64
_W = _D // 2
_N = 100000
_LIN = 256
_NC, _NS = 2, 16
_NW = _NC * _NS
_BPW = _B // _NW
_CH = 128
_NCH = _BPW // _CH


_CI = 4096
_QT = _CI // 4
_GRT = 25
_NV = _CI * _GRT

_TDN = (((0,), (0,)), ((), ()))


def _pack_bf16(tr):
    bits = lax.bitcast_convert_type(tr, jnp.uint32) + jnp.uint32(0x8000)
    lo = bits[:, :_W] >> 16
    hi = bits[:, _W:] & jnp.uint32(0xFFFF0000)
    return lax.bitcast_convert_type(hi | lo, jnp.float32)


def _repack_body(t_d, t_p, ident, out_d, out_p):
    iden = ident[...]
    for t_ref, o_ref in ((t_d, out_d), (t_p, out_p)):
        tr = lax.dot_general(t_ref[...], iden, _TDN,
                             preferred_element_type=jnp.float32)
        pk = _pack_bf16(tr)
        for q in range(4):
            o_ref[:, q * _W:(q + 1) * _W] = pk[q * _QT:(q + 1) * _QT]


_repack = pl.pallas_call(
    _repack_body,
    grid=(_GRT,),
    in_specs=[
        pl.BlockSpec((_D, _CI), lambda i: (0, i)),
        pl.BlockSpec((_D, _CI), lambda i: (0, i)),
        pl.BlockSpec((_D, _D), lambda i: (0, 0)),
    ],
    out_specs=[
        pl.BlockSpec((_QT, 4 * _W), lambda i: (i, 0)),
        pl.BlockSpec((_QT, 4 * _W), lambda i: (i, 0)),
    ],
    out_shape=[
        jax.ShapeDtypeStruct((_QT * _GRT, 4 * _W), jnp.float32),
        jax.ShapeDtypeStruct((_QT * _GRT, 4 * _W), jnp.float32),
    ],
)


_sc_mesh = plsc.VectorSubcoreMesh(core_axis_name="c", subcore_axis_name="s")


@functools.partial(
    pl.kernel,
    out_type=[
        jax.ShapeDtypeStruct((_B, _W), jnp.float32),
        jax.ShapeDtypeStruct((_B, _W), jnp.float32),
    ],
    mesh=_sc_mesh,
    compiler_params=pltpu.CompilerParams(
        use_tc_tiling_on_sc=False, needs_layout_passes=False),
    scratch_types=[
        pltpu.VMEM((2 * _BPW,), jnp.int32),
        pltpu.VMEM((_BPW,), jnp.int32),
        pltpu.VMEM((_BPW,), jnp.int32),
        pltpu.VMEM((2, _CH, _W), jnp.float32),
        pltpu.VMEM((2, _CH, _W), jnp.float32),
        pltpu.SemaphoreType.DMA,
    ],
)
def _sc_gather(flatidx_hbm, donor_tbl, proj_tbl, outd, outp,
               idx_iv, idx_d, idx_p, rows_d, rows_p, sem):
    wid = lax.axis_index("s") * _NC + lax.axis_index("c")
    base = wid * _BPW
    pltpu.sync_copy(flatidx_hbm.at[pl.ds(base, _BPW)],
                    idx_iv.at[pl.ds(0, _BPW)])
    pltpu.sync_copy(flatidx_hbm.at[pl.ds(_B + base, _BPW)],
                    idx_iv.at[pl.ds(_BPW, _BPW)])

    def remap(v):
        return ((v & jnp.int32(~(_CI - 1))) + 4 * (v & jnp.int32(_QT - 1))
                + ((v >> 10) & jnp.int32(3)))

    for j in range(_BPW // 16):
        sl = pl.ds(j * 16, 16)
        idx_d[sl] = remap(idx_iv[sl])
        idx_p[sl] = remap(idx_iv[pl.ds(_BPW + j * 16, 16)])
    copies = [None, None]

    def fire(j):
        buf = j % 2
        sl = pl.ds(j * _CH, _CH)
        copies[buf] = (
            pltpu.async_copy(donor_tbl.at[idx_d.at[sl]], rows_d.at[buf], sem),
            pltpu.async_copy(proj_tbl.at[idx_p.at[sl]], rows_p.at[buf], sem),
        )

    fire(0)
    for j in range(_NCH):
        if j + 1 < _NCH:
            fire(j + 1)
        buf = j % 2
        cp_d, cp_p = copies[buf]
        cp_d.wait()
        cp_p.wait()
        osl = pl.ds(base + j * _CH, _CH)
        pltpu.sync_copy(rows_d.at[buf], outd.at[osl])
        pltpu.sync_copy(rows_p.at[buf], outp.at[osl])


_B2 = _B // 2
_BLK = 1024


def _unpack_bf16(pk):
    u = lax.bitcast_convert_type(pk, jnp.uint32)
    lo = lax.bitcast_convert_type(u << 16, jnp.float32)
    hi = lax.bitcast_convert_type(u & jnp.uint32(0xFFFF0000), jnp.float32)
    return jnp.concatenate([lo, hi], axis=1)


def _mlp_body(d_ref, p_ref, w1a_ref, w1b_ref, b1_ref, w2_ref, b2_ref, o_ref):
    w1a = w1a_ref[...]
    w1b = w1b_ref[...]
    b1 = b1_ref[...]
    w2 = w2_ref[...]
    b2 = b2_ref[...]
    d = d_ref[...]
    p = p_ref[...]
    de = _unpack_bf16(d[:, :_W])
    do = _unpack_bf16(d[:, _W:])
    pe = _unpack_bf16(p[:, :_W])
    po = _unpack_bf16(p[:, _W:])
    he = jnp.dot(de, w1a, preferred_element_type=jnp.float32)
    he = he + jnp.dot(pe, w1b, preferred_element_type=jnp.float32)
    he = jnp.maximum(he + b1, 0.0)
    ho = jnp.dot(do, w1a, preferred_element_type=jnp.float32)
    ho = ho + jnp.dot(po, w1b, preferred_element_type=jnp.float32)
    ho = jnp.maximum(ho + b1, 0.0)
    o_ref[:, 0:1] = jnp.dot(he, w2, preferred_element_type=jnp.float32) + b2
    o_ref[:, 1:2] = jnp.dot(ho, w2, preferred_element_type=jnp.float32) + b2


_mlp = pl.pallas_call(
    _mlp_body,
    grid=(_B2 // _BLK,),
    in_specs=[
        pl.BlockSpec((_BLK, 2 * _W), lambda i: (i, 0)),
        pl.BlockSpec((_BLK, 2 * _W), lambda i: (i, 0)),
        pl.BlockSpec((_D, _LIN), lambda i: (0, 0)),
        pl.BlockSpec((_D, _LIN), lambda i: (0, 0)),
        pl.BlockSpec((1, _LIN), lambda i: (0, 0)),
        pl.BlockSpec((_LIN, 1), lambda i: (0, 0)),
        pl.BlockSpec((1, 1), lambda i: (0, 0)),
    ],
    out_specs=pl.BlockSpec((_BLK, 2), lambda i: (i, 0)),
    out_shape=jax.ShapeDtypeStruct((_B2, 2), jnp.float32),
)


@jax.jit
def kernel(input, emb_donor, emb_project, W1, b1, W2, b2):
    flatidx = input.astype(jnp.int32).T.reshape(2 * _B)
    ident = jnp.eye(_D, dtype=jnp.float32)
    packed_d, packed_p = _repack(emb_donor.T, emb_project.T, ident)
    lin_d = packed_d.reshape(_NV, _W)
    lin_p = packed_p.reshape(_NV, _W)
    rows_d, rows_p = _sc_gather(flatidx, lin_d, lin_p)
    w1t = W1.T
    y2 = _mlp(rows_d.reshape(_B2, 2 * _W), rows_p.reshape(_B2, 2 * _W),
              w1t[:_D], w1t[_D:],
              b1.reshape(1, _LIN), W2.T, b2.reshape(1, 1))
    return y2.reshape(_B, 1)


# speedup vs baseline: 1.1595x; 1.1595x over previous
import functools

import jax
import jax.numpy as jnp
from jax import lax
from jax.experimental import pallas as pl
from jax.experimental.pallas import tpu as pltpu
from jax.experimental.pallas import tpu_sc as plsc

_B = 16384
_D = 64
_N = 100000
_S = _N // 2
_LIN = 256
_NC, _NS = 2, 16
_NW = _NC * _NS
_BPW = _B // _NW
_CH = 128
_NCH = _BPW // _CH


_CT = 2048
_GRT = 25
_NV = 2 * _CT * _GRT

_TDN = (((0,), (0,)), ((), ()))


def _repack_body(t_ref, ident, out_ref):
    tr = lax.dot_general(t_ref[...], ident[...], _TDN,
                         preferred_element_type=jnp.float32)
    out_ref[:, :_D] = tr[:_CT]
    out_ref[:, _D:] = tr[_CT:]


_repack = pl.pallas_call(
    _repack_body,
    grid=(_GRT,),
    in_specs=[
        pl.BlockSpec((_D, 2 * _CT), lambda i: (0, i)),
        pl.BlockSpec((_D, _D), lambda i: (0, 0)),
    ],
    out_specs=pl.BlockSpec((_CT, 2 * _D), lambda i: (i, 0)),
    out_shape=jax.ShapeDtypeStruct((_CT * _GRT, 2 * _D), jnp.float32),
)


_sc_mesh = plsc.VectorSubcoreMesh(core_axis_name="c", subcore_axis_name="s")


@functools.partial(
    pl.kernel,
    out_type=jax.ShapeDtypeStruct((_B, _D), jnp.float32),
    mesh=_sc_mesh,
    compiler_params=pltpu.CompilerParams(
        use_tc_tiling_on_sc=False, needs_layout_passes=False),
    scratch_types=[
        pltpu.VMEM((_BPW,), jnp.int32),
        pltpu.VMEM((_BPW,), jnp.int32),
        pltpu.VMEM((2, _CH, _D), jnp.float32),
        pltpu.SemaphoreType.DMA,
    ],
)
def _sc_gather(idx_hbm, tbl, out, idx_iv, idx_v, rows, sem):
    wid = lax.axis_index("s") * _NC + lax.axis_index("c")
    base = wid * _BPW
    pltpu.sync_copy(idx_hbm.at[pl.ds(base, _BPW)], idx_iv)

    def remap(v):
        blk = jnp.bitwise_and(v, 2 * _CT - 1)
        return (v - blk) + jnp.where(blk < _CT, 2 * blk, 2 * blk - (2 * _CT - 1))

    for j in range(_BPW // 16):
        sl = pl.ds(j * 16, 16)
        idx_v[sl] = remap(idx_iv[sl])
    copies = [None, None]

    def fire(j):
        buf = j % 2
        sl = pl.ds(j * _CH, _CH)
        copies[buf] = pltpu.async_copy(
            tbl.at[idx_v.at[sl]], rows.at[buf], sem)

    fire(0)
    for j in range(_NCH):
        if j + 1 < _NCH:
            fire(j + 1)
        buf = j % 2
        copies[buf].wait()
        pltpu.sync_copy(rows.at[buf], out.at[pl.ds(base + j * _CH, _CH)])


_B2 = _B // 2
_BLK = 1024


def _mlp_body(d_ref, p_ref, w1a_ref, w1b_ref, b1_ref, w2_ref, b2_ref, o_ref):
    w1a = w1a_ref[...]
    w1b = w1b_ref[...]
    b1 = b1_ref[...]
    w2 = w2_ref[...]
    b2 = b2_ref[...]
    de, do = d_ref[:, :_D], d_ref[:, _D:]
    pe, po = p_ref[:, :_D], p_ref[:, _D:]
    he = jnp.dot(de, w1a, preferred_element_type=jnp.float32)
    he = he + jnp.dot(pe, w1b, preferred_element_type=jnp.float32)
    he = jnp.maximum(he + b1, 0.0)
    ho = jnp.dot(do, w1a, preferred_element_type=jnp.float32)
    ho = ho + jnp.dot(po, w1b, preferred_element_type=jnp.float32)
    ho = jnp.maximum(ho + b1, 0.0)
    o_ref[:, 0:1] = jnp.dot(he, w2, preferred_element_type=jnp.float32) + b2
    o_ref[:, 1:2] = jnp.dot(ho, w2, preferred_element_type=jnp.float32) + b2


_mlp = pl.pallas_call(
    _mlp_body,
    grid=(_B2 // _BLK,),
    in_specs=[
        pl.BlockSpec((_BLK, 2 * _D), lambda i: (i, 0)),
        pl.BlockSpec((_BLK, 2 * _D), lambda i: (i, 0)),
        pl.BlockSpec((_D, _LIN), lambda i: (0, 0)),
        pl.BlockSpec((_D, _LIN), lambda i: (0, 0)),
        pl.BlockSpec((1, _LIN), lambda i: (0, 0)),
        pl.BlockSpec((_LIN, 1), lambda i: (0, 0)),
        pl.BlockSpec((1, 1), lambda i: (0, 0)),
    ],
    out_specs=pl.BlockSpec((_BLK, 2), lambda i: (i, 0)),
    out_shape=jax.ShapeDtypeStruct((_B2, 2), jnp.float32),
)


@jax.jit
def kernel(input, emb_donor, emb_project, W1, b1, W2, b2):
    idx2 = input.astype(jnp.int32).T
    ident = jnp.eye(_D, dtype=jnp.float32)
    packed_d = _repack(emb_donor.T, ident)
    rows_d = _sc_gather(idx2[0], packed_d.reshape(_NV, _D))
    packed_p = _repack(emb_project.T, ident)
    rows_p = _sc_gather(idx2[1], packed_p.reshape(_NV, _D))
    w1t = W1.T
    y2 = _mlp(rows_d.reshape(_B2, 2 * _D), rows_p.reshape(_B2, 2 * _D),
              w1t[:_D], w1t[_D:],
              b1.reshape(1, _LIN), W2.T, b2.reshape(1, 1))
    return y2.reshape(_B, 1)


# speedup vs baseline: 1.3367x; 1.1528x over previous
import functools

import jax
import jax.numpy as jnp
from jax import lax
from jax.experimental import pallas as pl
from jax.experimental.pallas import tpu as pltpu
from jax.experimental.pallas import tpu_sc as plsc

_B = 16384
_D = 64
_N = 100000
_S = _N // 2
_LIN = 256
_NC, _NS = 2, 16
_NW = _NC * _NS
_BPW = _B // _NW
_CH = 128
_NCH = _BPW // _CH


_CT = 2048
_GRT = 25
_NV = 2 * _CT * _GRT

_TDN = (((0,), (0,)), ((), ()))


def _repack_body(t_d, t_p, ident, out_d, out_p):
    iden = ident[...]
    tr_d = lax.dot_general(t_d[...], iden, _TDN,
                           preferred_element_type=jnp.float32)
    out_d[:, :_D] = tr_d[:_CT]
    out_d[:, _D:] = tr_d[_CT:]
    tr_p = lax.dot_general(t_p[...], iden, _TDN,
                           preferred_element_type=jnp.float32)
    out_p[:, :_D] = tr_p[:_CT]
    out_p[:, _D:] = tr_p[_CT:]


_repack = pl.pallas_call(
    _repack_body,
    grid=(_GRT,),
    in_specs=[
        pl.BlockSpec((_D, 2 * _CT), lambda i: (0, i)),
        pl.BlockSpec((_D, 2 * _CT), lambda i: (0, i)),
        pl.BlockSpec((_D, _D), lambda i: (0, 0)),
    ],
    out_specs=[
        pl.BlockSpec((_CT, 2 * _D), lambda i: (i, 0)),
        pl.BlockSpec((_CT, 2 * _D), lambda i: (i, 0)),
    ],
    out_shape=[
        jax.ShapeDtypeStruct((_CT * _GRT, 2 * _D), jnp.float32),
        jax.ShapeDtypeStruct((_CT * _GRT, 2 * _D), jnp.float32),
    ],
)


_sc_mesh = plsc.VectorSubcoreMesh(core_axis_name="c", subcore_axis_name="s")


@functools.partial(
    pl.kernel,
    out_type=[
        jax.ShapeDtypeStruct((_B, _D), jnp.float32),
        jax.ShapeDtypeStruct((_B, _D), jnp.float32),
    ],
    mesh=_sc_mesh,
    compiler_params=pltpu.CompilerParams(
        use_tc_tiling_on_sc=False, needs_layout_passes=False),
    scratch_types=[
        pltpu.VMEM((2 * _BPW,), jnp.int32),
        pltpu.VMEM((_BPW,), jnp.int32),
        pltpu.VMEM((_BPW,), jnp.int32),
        pltpu.VMEM((2, _CH, _D), jnp.float32),
        pltpu.VMEM((2, _CH, _D), jnp.float32),
        pltpu.SemaphoreType.DMA,
    ],
)
def _sc_gather(flatidx_hbm, donor_tbl, proj_tbl, outd, outp,
               idx_iv, idx_d, idx_p, rows_d, rows_p, sem):
    wid = lax.axis_index("s") * _NC + lax.axis_index("c")
    base = wid * _BPW
    pltpu.sync_copy(flatidx_hbm.at[pl.ds(base, _BPW)],
                    idx_iv.at[pl.ds(0, _BPW)])
    pltpu.sync_copy(flatidx_hbm.at[pl.ds(_B + base, _BPW)],
                    idx_iv.at[pl.ds(_BPW, _BPW)])

    def remap(v):
        blk = jnp.bitwise_and(v, 2 * _CT - 1)
        return (v - blk) + jnp.where(blk < _CT, 2 * blk, 2 * blk - (2 * _CT - 1))

    for j in range(_BPW // 16):
        sl = pl.ds(j * 16, 16)
        idx_d[sl] = remap(idx_iv[sl])
        idx_p[sl] = remap(idx_iv[pl.ds(_BPW + j * 16, 16)])
    copies = [None, None]

    def fire(j):
        buf = j % 2
        sl = pl.ds(j * _CH, _CH)
        copies[buf] = (
            pltpu.async_copy(donor_tbl.at[idx_d.at[sl]], rows_d.at[buf], sem),
            pltpu.async_copy(proj_tbl.at[idx_p.at[sl]], rows_p.at[buf], sem),
        )

    fire(0)
    for j in range(_NCH):
        if j + 1 < _NCH:
            fire(j + 1)
        buf = j % 2
        cp_d, cp_p = copies[buf]
        cp_d.wait()
        cp_p.wait()
        osl = pl.ds(base + j * _CH, _CH)
        pltpu.sync_copy(rows_d.at[buf], outd.at[osl])
        pltpu.sync_copy(rows_p.at[buf], outp.at[osl])


_B2 = _B // 2
_BLK = 1024


def _mlp_body(d_ref, p_ref, w1a_ref, w1b_ref, b1_ref, w2_ref, b2_ref, o_ref):
    w1a = w1a_ref[...]
    w1b = w1b_ref[...]
    b1 = b1_ref[...]
    w2 = w2_ref[...]
    b2 = b2_ref[...]
    de, do = d_ref[:, :_D], d_ref[:, _D:]
    pe, po = p_ref[:, :_D], p_ref[:, _D:]
    he = jnp.dot(de, w1a, preferred_element_type=jnp.float32)
    he = he + jnp.dot(pe, w1b, preferred_element_type=jnp.float32)
    he = jnp.maximum(he + b1, 0.0)
    ho = jnp.dot(do, w1a, preferred_element_type=jnp.float32)
    ho = ho + jnp.dot(po, w1b, preferred_element_type=jnp.float32)
    ho = jnp.maximum(ho + b1, 0.0)
    o_ref[:, 0:1] = jnp.dot(he, w2, preferred_element_type=jnp.float32) + b2
    o_ref[:, 1:2] = jnp.dot(ho, w2, preferred_element_type=jnp.float32) + b2


_mlp = pl.pallas_call(
    _mlp_body,
    grid=(_B2 // _BLK,),
    in_specs=[
        pl.BlockSpec((_BLK, 2 * _D), lambda i: (i, 0)),
        pl.BlockSpec((_BLK, 2 * _D), lambda i: (i, 0)),
        pl.BlockSpec((_D, _LIN), lambda i: (0, 0)),
        pl.BlockSpec((_D, _LIN), lambda i: (0, 0)),
        pl.BlockSpec((1, _LIN), lambda i: (0, 0)),
        pl.BlockSpec((_LIN, 1), lambda i: (0, 0)),
        pl.BlockSpec((1, 1), lambda i: (0, 0)),
    ],
    out_specs=pl.BlockSpec((_BLK, 2), lambda i: (i, 0)),
    out_shape=jax.ShapeDtypeStruct((_B2, 2), jnp.float32),
)


@jax.jit
def kernel(input, emb_donor, emb_project, W1, b1, W2, b2):
    flatidx = input.astype(jnp.int32).T.reshape(2 * _B)
    ident = jnp.eye(_D, dtype=jnp.float32)
    packed_d, packed_p = _repack(emb_donor.T, emb_project.T, ident)
    lin_d = packed_d.reshape(_NV, _D)
    lin_p = packed_p.reshape(_NV, _D)
    rows_d, rows_p = _sc_gather(flatidx, lin_d, lin_p)
    w1t = W1.T
    y2 = _mlp(rows_d.reshape(_B2, 2 * _D), rows_p.reshape(_B2, 2 * _D),
              w1t[:_D], w1t[_D:],
              b1.reshape(1, _LIN), W2.T, b2.reshape(1, 1))
    return y2.reshape(_B, 1)


# speedup vs baseline: 1.3984x; 1.0461x over previous
import functools

import jax
import jax.numpy as jnp
from jax import lax
from jax.experimental import pallas as pl
from jax.experimental.pallas import tpu as pltpu
from jax.experimental.pallas import tpu_sc as plsc

_B = 16384
_D = 64
_N = 100000
_S = _N // 2
_LIN = 256
_NC, _NS = 2, 16
_NW = _NC * _NS
_BPW = _B // _NW
_CH = 128
_NCH = _BPW // _CH


_CT = 4096
_GRT = 13
_NV = 2 * _CT * _GRT

_TDN = (((0,), (0,)), ((), ()))


def _repack_body(t_d, t_p, ident, out_d, out_p):
    iden = ident[...]
    tr_d = lax.dot_general(t_d[...], iden, _TDN,
                           preferred_element_type=jnp.float32)
    out_d[:, :_D] = tr_d[:_CT]
    out_d[:, _D:] = tr_d[_CT:]
    tr_p = lax.dot_general(t_p[...], iden, _TDN,
                           preferred_element_type=jnp.float32)
    out_p[:, :_D] = tr_p[:_CT]
    out_p[:, _D:] = tr_p[_CT:]


_repack = pl.pallas_call(
    _repack_body,
    grid=(_GRT,),
    in_specs=[
        pl.BlockSpec((_D, 2 * _CT), lambda i: (0, i)),
        pl.BlockSpec((_D, 2 * _CT), lambda i: (0, i)),
        pl.BlockSpec((_D, _D), lambda i: (0, 0)),
    ],
    out_specs=[
        pl.BlockSpec((_CT, 2 * _D), lambda i: (i, 0)),
        pl.BlockSpec((_CT, 2 * _D), lambda i: (i, 0)),
    ],
    out_shape=[
        jax.ShapeDtypeStruct((_CT * _GRT, 2 * _D), jnp.float32),
        jax.ShapeDtypeStruct((_CT * _GRT, 2 * _D), jnp.float32),
    ],
)


_sc_mesh = plsc.VectorSubcoreMesh(core_axis_name="c", subcore_axis_name="s")


@functools.partial(
    pl.kernel,
    out_type=[
        jax.ShapeDtypeStruct((_B, _D), jnp.float32),
        jax.ShapeDtypeStruct((_B, _D), jnp.float32),
    ],
    mesh=_sc_mesh,
    compiler_params=pltpu.CompilerParams(
        use_tc_tiling_on_sc=False, needs_layout_passes=False),
    scratch_types=[
        pltpu.VMEM((2 * _BPW,), jnp.int32),
        pltpu.VMEM((_BPW,), jnp.int32),
        pltpu.VMEM((_BPW,), jnp.int32),
        pltpu.VMEM((2, _CH, _D), jnp.float32),
        pltpu.VMEM((2, _CH, _D), jnp.float32),
        pltpu.SemaphoreType.DMA,
    ],
)
def _sc_gather(flatidx_hbm, donor_tbl, proj_tbl, outd, outp,
               idx_iv, idx_d, idx_p, rows_d, rows_p, sem):
    wid = lax.axis_index("s") * _NC + lax.axis_index("c")
    base = wid * _BPW
    pltpu.sync_copy(flatidx_hbm.at[pl.ds(base, _BPW)],
                    idx_iv.at[pl.ds(0, _BPW)])
    pltpu.sync_copy(flatidx_hbm.at[pl.ds(_B + base, _BPW)],
                    idx_iv.at[pl.ds(_BPW, _BPW)])

    def remap(v):
        blk = jnp.bitwise_and(v, 2 * _CT - 1)
        return (v - blk) + jnp.where(blk < _CT, 2 * blk, 2 * blk - (2 * _CT - 1))

    for j in range(_BPW // 16):
        sl = pl.ds(j * 16, 16)
        idx_d[sl] = remap(idx_iv[sl])
        idx_p[sl] = remap(idx_iv[pl.ds(_BPW + j * 16, 16)])
    copies = [None, None]

    def fire(j):
        buf = j % 2
        sl = pl.ds(j * _CH, _CH)
        copies[buf] = (
            pltpu.async_copy(donor_tbl.at[idx_d.at[sl]], rows_d.at[buf], sem),
            pltpu.async_copy(proj_tbl.at[idx_p.at[sl]], rows_p.at[buf], sem),
        )

    fire(0)
    for j in range(_NCH):
        if j + 1 < _NCH:
            fire(j + 1)
        buf = j % 2
        cp_d, cp_p = copies[buf]
        cp_d.wait()
        cp_p.wait()
        osl = pl.ds(base + j * _CH, _CH)
        pltpu.sync_copy(rows_d.at[buf], outd.at[osl])
        pltpu.sync_copy(rows_p.at[buf], outp.at[osl])


_B2 = _B // 2
_BLK = 1024


def _mlp_body(d_ref, p_ref, w1a_ref, w1b_ref, b1_ref, w2_ref, b2_ref, o_ref):
    w1a = w1a_ref[...]
    w1b = w1b_ref[...]
    b1 = b1_ref[...]
    w2 = w2_ref[...]
    b2 = b2_ref[...]
    de, do = d_ref[:, :_D], d_ref[:, _D:]
    pe, po = p_ref[:, :_D], p_ref[:, _D:]
    he = jnp.dot(de, w1a, preferred_element_type=jnp.float32)
    he = he + jnp.dot(pe, w1b, preferred_element_type=jnp.float32)
    he = jnp.maximum(he + b1, 0.0)
    ho = jnp.dot(do, w1a, preferred_element_type=jnp.float32)
    ho = ho + jnp.dot(po, w1b, preferred_element_type=jnp.float32)
    ho = jnp.maximum(ho + b1, 0.0)
    o_ref[:, 0:1] = jnp.dot(he, w2, preferred_element_type=jnp.float32) + b2
    o_ref[:, 1:2] = jnp.dot(ho, w2, preferred_element_type=jnp.float32) + b2


_mlp = pl.pallas_call(
    _mlp_body,
    grid=(_B2 // _BLK,),
    in_specs=[
        pl.BlockSpec((_BLK, 2 * _D), lambda i: (i, 0)),
        pl.BlockSpec((_BLK, 2 * _D), lambda i: (i, 0)),
        pl.BlockSpec((_D, _LIN), lambda i: (0, 0)),
        pl.BlockSpec((_D, _LIN), lambda i: (0, 0)),
        pl.BlockSpec((1, _LIN), lambda i: (0, 0)),
        pl.BlockSpec((_LIN, 1), lambda i: (0, 0)),
        pl.BlockSpec((1, 1), lambda i: (0, 0)),
    ],
    out_specs=pl.BlockSpec((_BLK, 2), lambda i: (i, 0)),
    out_shape=jax.ShapeDtypeStruct((_B2, 2), jnp.float32),
)


@jax.jit
def kernel(input, emb_donor, emb_project, W1, b1, W2, b2):
    flatidx = input.astype(jnp.int32).T.reshape(2 * _B)
    ident = jnp.eye(_D, dtype=jnp.float32)
    packed_d, packed_p = _repack(emb_donor.T, emb_project.T, ident)
    lin_d = packed_d.reshape(_NV, _D)
    lin_p = packed_p.reshape(_NV, _D)
    rows_d, rows_p = _sc_gather(flatidx, lin_d, lin_p)
    w1t = W1.T
    y2 = _mlp(rows_d.reshape(_B2, 2 * _D), rows_p.reshape(_B2, 2 * _D),
              w1t[:_D], w1t[_D:],
              b1.reshape(1, _LIN), W2.T, b2.reshape(1, 1))
    return y2.reshape(_B, 1)
